# baseline (speedup 1.0000x reference)
import functools

import jax
import jax.numpy as jnp
from jax import lax
from jax.experimental import pallas as pl
from jax.experimental.pallas import tpu as pltpu
from jax.experimental.pallas import tpu_sc as plsc

N = 10000
E = 320000
D = 128
P = 512
MLP_DIM = 256
NACT = 10
LAYERS = 3

NC = 2
NS = 16
CH = 64
NBUF = 4
SUP = 40
NSUP = 8
NCHUNK = SUP * NSUP
EPT_PAD = NCHUNK * CH
NPAD = N + 16
RPT = 624
RPT_LAST = N - (NS - 1) * RPT


def _sc_agg(hs, hf, gidx, sidx, zeros):
  mesh = plsc.VectorSubcoreMesh(core_axis_name="c", subcore_axis_name="s")

  @functools.partial(
      pl.kernel,
      out_type=[jax.ShapeDtypeStruct((N, D), jnp.float32)] * 2,
      mesh=mesh,
      scratch_types=[
          pltpu.VMEM((SUP, CH), jnp.int32),
          pltpu.VMEM((SUP, CH), jnp.int32),
          [pltpu.VMEM((CH, D), jnp.float32)] * NBUF,
          [pltpu.SemaphoreType.DMA] * NBUF,
          [pltpu.SemaphoreType.DMA] * NBUF,
          pltpu.VMEM_SHARED((NPAD, D), jnp.float32),
      ],
  )
  def k(hs_hbm, hf_hbm, g_hbm, s_hbm, z_hbm, aggs_hbm, aggf_hbm,
        gv, sv, bufs, gsems, ssems, agg_sh):
    c = lax.axis_index("c")
    s = lax.axis_index("s")
    base = s * RPT

    @pl.when(s < NS - 1)
    def _():
      pltpu.sync_copy(z_hbm.at[pl.ds(0, RPT)], agg_sh.at[pl.ds(base, RPT)])

    @pl.when(s == NS - 1)
    def _():
      pltpu.sync_copy(z_hbm.at[pl.ds(0, RPT_LAST)],
                      agg_sh.at[pl.ds(base, RPT_LAST)])

    plsc.subcore_barrier()

    def run(h_hbm):
      def superstep(sup, _):
        pltpu.sync_copy(g_hbm.at[c, s, sup], gv)
        pltpu.sync_copy(s_hbm.at[c, s, sup], sv)
        for b in range(NBUF):
          pltpu.async_copy(h_hbm.at[gv.at[b]], bufs[b], gsems[b])

        def round_body(i, _):
          j0 = i * NBUF
          for b in range(NBUF):
            pltpu.make_async_copy(h_hbm.at[gv.at[j0 + b]],
                                  bufs[b], gsems[b]).wait()
            pltpu.async_copy(bufs[b], agg_sh.at[sv.at[j0 + b]], ssems[b],
                             add=True)
          for b in range(NBUF):
            pltpu.make_async_copy(bufs[b], agg_sh.at[sv.at[j0 + b]],
                                  ssems[b]).wait()
            pltpu.async_copy(h_hbm.at[gv.at[j0 + NBUF + b]], bufs[b],
                             gsems[b])
          return 0

        lax.fori_loop(0, SUP // NBUF - 1, round_body, 0)
        j0 = SUP - NBUF
        for b in range(NBUF):
          pltpu.make_async_copy(h_hbm.at[gv.at[j0 + b]],
                                bufs[b], gsems[b]).wait()
          pltpu.async_copy(bufs[b], agg_sh.at[sv.at[j0 + b]], ssems[b],
                           add=True)
        for b in range(NBUF):
          pltpu.make_async_copy(bufs[b], agg_sh.at[sv.at[j0 + b]],
                                ssems[b]).wait()
        return 0

      lax.fori_loop(0, NSUP, superstep, 0)

    @pl.when(c == 0)
    def _():
      run(hs_hbm)

    @pl.when(c == 1)
    def _():
      run(hf_hbm)

    plsc.subcore_barrier()

    def copy_out(out_hbm):
      @pl.when(s < NS - 1)
      def _():
        pltpu.sync_copy(agg_sh.at[pl.ds(base, RPT)],
                        out_hbm.at[pl.ds(base, RPT)])

      @pl.when(s == NS - 1)
      def _():
        pltpu.sync_copy(agg_sh.at[pl.ds(base, RPT_LAST)],
                        out_hbm.at[pl.ds(base, RPT_LAST)])

    @pl.when(c == 0)
    def _():
      copy_out(aggs_hbm)

    @pl.when(c == 1)
    def _():
      copy_out(aggf_hbm)

  return k(hs, hf, gidx, sidx, zeros)


_BLK = 1000


def _tc_dense_body(aggs_ref, hs_ref, aggf_ref, hf_ref,
                   wns, wss, bs1, wnf, wsf, bf1, os_ref, of_ref):
  os_ref[...] = jnp.maximum(
      jnp.dot(aggs_ref[...], wns[...], preferred_element_type=jnp.float32)
      + jnp.dot(hs_ref[...], wss[...], preferred_element_type=jnp.float32)
      + bs1[...], 0.0)
  of_ref[...] = jnp.maximum(
      jnp.dot(aggf_ref[...], wnf[...], preferred_element_type=jnp.float32)
      + jnp.dot(hf_ref[...], wsf[...], preferred_element_type=jnp.float32)
      + bf1[...], 0.0)


def _tc_dense(aggs, hs, aggf, hf, wns, wss, bs1, wnf, wsf, bf1):
  row_spec = pl.BlockSpec((_BLK, D), lambda i: (i, 0))
  w_spec = pl.BlockSpec((D, D), lambda i: (0, 0))
  b_spec = pl.BlockSpec((1, D), lambda i: (0, 0))
  return pl.pallas_call(
      _tc_dense_body,
      grid=(N // _BLK,),
      in_specs=[row_spec, row_spec, row_spec, row_spec,
                w_spec, w_spec, b_spec, w_spec, w_spec, b_spec],
      out_specs=[row_spec, row_spec],
      out_shape=[jax.ShapeDtypeStruct((N, D), jnp.float32)] * 2,
  )(aggs, hs, aggf, hf, wns, wss, bs1, wnf, wsf, bf1)


_PPT = P // NS


def _sc_po_gather(hs, hf, pos):
  mesh = plsc.VectorSubcoreMesh(core_axis_name="c", subcore_axis_name="s")

  @functools.partial(
      pl.kernel,
      out_type=[jax.ShapeDtypeStruct((P, D), jnp.float32)] * 2,
      mesh=mesh,
      scratch_types=[
          pltpu.VMEM((_PPT,), jnp.int32),
          pltpu.VMEM((_PPT, D), jnp.float32),
          pltpu.SemaphoreType.DMA,
      ],
  )
  def k(hs_hbm, hf_hbm, pos_hbm, embs_hbm, embf_hbm, pidx, rows, sem):
    c = lax.axis_index("c")
    s = lax.axis_index("s")
    base = s * _PPT
    pltpu.sync_copy(pos_hbm.at[pl.ds(base, _PPT)], pidx)

    @pl.when(c == 0)
    def _():
      pltpu.async_copy(hs_hbm.at[pidx], rows, sem).wait()
      pltpu.sync_copy(rows, embs_hbm.at[pl.ds(base, _PPT)])

    @pl.when(c == 1)
    def _():
      pltpu.async_copy(hf_hbm.at[pidx], rows, sem).wait()
      pltpu.sync_copy(rows, embf_hbm.at[pl.ds(base, _PPT)])

  return k(hs, hf, pos)


def _tc_mlp_body(es_ref, ef_ref, w1s, w1f, b1r, w2r, b2r, w3r, b3r, out_ref):
  h = jnp.maximum(
      jnp.dot(es_ref[...], w1s[...], preferred_element_type=jnp.float32)
      + jnp.dot(ef_ref[...], w1f[...], preferred_element_type=jnp.float32)
      + b1r[...], 0.0)
  h = jnp.maximum(
      jnp.dot(h, w2r[...], preferred_element_type=jnp.float32) + b2r[...], 0.0)
  out_ref[...] = (
      jnp.dot(h, w3r[...], preferred_element_type=jnp.float32) + b3r[...])


def _tc_mlp(embs, embf, w1s, w1f, b1, w2, b2, w3p, b3p):
  return pl.pallas_call(
      _tc_mlp_body,
      out_shape=jax.ShapeDtypeStruct((P, 128), jnp.float32),
  )(embs, embf, w1s, w1f, b1, w2, b2, w3p, b3p)


def kernel(x, edge_index, POs, Wn_s, Wself_s, b_s, Wn_f, Wself_f, b_f,
           W1, b1, W2, b2, W3, b3):
  src = edge_index[0]
  dst = edge_index[1]
  pad = NS * EPT_PAD - E
  gpad = jnp.zeros((pad,), jnp.int32)
  spad = jnp.full((pad,), N, jnp.int32)
  gidx = jnp.stack([jnp.concatenate([src, gpad]),
                    jnp.concatenate([dst, gpad])]).reshape(NC, NS, NSUP, SUP, CH)
  sidx = jnp.stack([jnp.concatenate([dst, spad]),
                    jnp.concatenate([src, spad])]).reshape(NC, NS, NSUP, SUP, CH)
  zeros = jnp.zeros((RPT_LAST, D), jnp.float32)

  hs = x
  hf = x
  for l in range(LAYERS):
    aggs, aggf = _sc_agg(hs, hf, gidx, sidx, zeros)
    hs, hf = _tc_dense(aggs, hs, aggf, hf,
                       Wn_s[l], Wself_s[l], b_s[l].reshape(1, D),
                       Wn_f[l], Wself_f[l], b_f[l].reshape(1, D))

  embs, embf = _sc_po_gather(hs, hf, POs)
  w3p = jnp.zeros((MLP_DIM, 128), jnp.float32).at[:, :NACT].set(W3)
  b3p = jnp.zeros((1, 128), jnp.float32).at[:, :NACT].set(b3.reshape(1, NACT))
  y = _tc_mlp(embs, embf, W1[:D], W1[D:], b1.reshape(1, MLP_DIM),
              W2, b2.reshape(1, MLP_DIM), w3p, b3p)
  return y[:, :NACT]


# speedup vs baseline: 2.6657x; 2.6657x over previous
import functools

import jax
import jax.numpy as jnp
from jax import lax
from jax.experimental import pallas as pl
from jax.experimental.pallas import tpu as pltpu
from jax.experimental.pallas import tpu_sc as plsc

N = 10000
E = 320000
D = 128
P = 512
MLP_DIM = 256
NACT = 10
LAYERS = 3

NC = 2
NS = 16
CH = 64
NBUF = 4
SUP = 40
NSUP = 8
NCHUNK = SUP * NSUP
EPT = E // NS
EPT_PAD = NCHUNK * CH
NPAD = N + 256
RPT = 624
RPT_LAST = N - (NS - 1) * RPT


def _sc_agg(hs, hf, gidx, sidx, zeros):
  mesh = plsc.VectorSubcoreMesh(core_axis_name="c", subcore_axis_name="s")

  @functools.partial(
      pl.kernel,
      out_type=[jax.ShapeDtypeStruct((N, D), jnp.float32)] * 2,
      mesh=mesh,
      scratch_types=[
          pltpu.VMEM((SUP, CH), jnp.int32),
          pltpu.VMEM((SUP, CH), jnp.int32),
          [pltpu.VMEM((CH, D), jnp.float32)] * NBUF,
          [pltpu.SemaphoreType.DMA] * NBUF,
          [pltpu.SemaphoreType.DMA] * NBUF,
          pltpu.VMEM_SHARED((NPAD, D), jnp.float32),
      ],
  )
  def k(hs_hbm, hf_hbm, g_hbm, s_hbm, z_hbm, aggs_hbm, aggf_hbm,
        gv, sv, bufs, gsems, ssems, agg_sh):
    c = lax.axis_index("c")
    s = lax.axis_index("s")
    base = s * RPT

    @pl.when(s < NS - 1)
    def _():
      pltpu.sync_copy(z_hbm.at[pl.ds(0, RPT)], agg_sh.at[pl.ds(base, RPT)])

    @pl.when(s == NS - 1)
    def _():
      pltpu.sync_copy(z_hbm.at[pl.ds(0, RPT_LAST)],
                      agg_sh.at[pl.ds(base, RPT_LAST)])

    plsc.subcore_barrier()

    def run(h_hbm):
      def superstep(sup, _):
        pltpu.sync_copy(g_hbm.at[c, s, sup], gv)
        pltpu.sync_copy(s_hbm.at[c, s, sup], sv)
        for b in range(NBUF):
          pltpu.async_copy(h_hbm.at[gv.at[b]], bufs[b], gsems[b])

        def round_body(i, _):
          j0 = i * NBUF
          for b in range(NBUF):
            pltpu.make_async_copy(h_hbm.at[gv.at[j0 + b]],
                                  bufs[b], gsems[b]).wait()
            pltpu.async_copy(bufs[b], agg_sh.at[sv.at[j0 + b]], ssems[b],
                             add=True)
          for b in range(NBUF):
            pltpu.make_async_copy(bufs[b], agg_sh.at[sv.at[j0 + b]],
                                  ssems[b]).wait()
            pltpu.async_copy(h_hbm.at[gv.at[j0 + NBUF + b]], bufs[b],
                             gsems[b])
          return 0

        lax.fori_loop(0, SUP // NBUF - 1, round_body, 0)
        j0 = SUP - NBUF
        for b in range(NBUF):
          pltpu.make_async_copy(h_hbm.at[gv.at[j0 + b]],
                                bufs[b], gsems[b]).wait()
          pltpu.async_copy(bufs[b], agg_sh.at[sv.at[j0 + b]], ssems[b],
                           add=True)
        for b in range(NBUF):
          pltpu.make_async_copy(bufs[b], agg_sh.at[sv.at[j0 + b]],
                                ssems[b]).wait()
        return 0

      lax.fori_loop(0, NSUP, superstep, 0)

    @pl.when(c == 0)
    def _():
      run(hs_hbm)

    @pl.when(c == 1)
    def _():
      run(hf_hbm)

    plsc.subcore_barrier()

    def copy_out(out_hbm):
      @pl.when(s < NS - 1)
      def _():
        pltpu.sync_copy(agg_sh.at[pl.ds(base, RPT)],
                        out_hbm.at[pl.ds(base, RPT)])

      @pl.when(s == NS - 1)
      def _():
        pltpu.sync_copy(agg_sh.at[pl.ds(base, RPT_LAST)],
                        out_hbm.at[pl.ds(base, RPT_LAST)])

    @pl.when(c == 0)
    def _():
      copy_out(aggs_hbm)

    @pl.when(c == 1)
    def _():
      copy_out(aggf_hbm)

  return k(hs, hf, gidx, sidx, zeros)


_BLK = 1000


def _tc_dense_body(aggs_ref, hs_ref, aggf_ref, hf_ref,
                   wns, wss, bs1, wnf, wsf, bf1, os_ref, of_ref):
  os_ref[...] = jnp.maximum(
      jnp.dot(aggs_ref[...], wns[...], preferred_element_type=jnp.float32)
      + jnp.dot(hs_ref[...], wss[...], preferred_element_type=jnp.float32)
      + bs1[...], 0.0)
  of_ref[...] = jnp.maximum(
      jnp.dot(aggf_ref[...], wnf[...], preferred_element_type=jnp.float32)
      + jnp.dot(hf_ref[...], wsf[...], preferred_element_type=jnp.float32)
      + bf1[...], 0.0)


def _tc_dense(aggs, hs, aggf, hf, wns, wss, bs1, wnf, wsf, bf1):
  row_spec = pl.BlockSpec((_BLK, D), lambda i: (i, 0))
  w_spec = pl.BlockSpec((D, D), lambda i: (0, 0))
  b_spec = pl.BlockSpec((1, D), lambda i: (0, 0))
  return pl.pallas_call(
      _tc_dense_body,
      grid=(N // _BLK,),
      in_specs=[row_spec, row_spec, row_spec, row_spec,
                w_spec, w_spec, b_spec, w_spec, w_spec, b_spec],
      out_specs=[row_spec, row_spec],
      out_shape=[jax.ShapeDtypeStruct((N, D), jnp.float32)] * 2,
  )(aggs, hs, aggf, hf, wns, wss, bs1, wnf, wsf, bf1)


_PPT = P // NS


def _sc_po_gather(hs, hf, pos):
  mesh = plsc.VectorSubcoreMesh(core_axis_name="c", subcore_axis_name="s")

  @functools.partial(
      pl.kernel,
      out_type=[jax.ShapeDtypeStruct((P, D), jnp.float32)] * 2,
      mesh=mesh,
      scratch_types=[
          pltpu.VMEM((_PPT,), jnp.int32),
          pltpu.VMEM((_PPT, D), jnp.float32),
          pltpu.SemaphoreType.DMA,
      ],
  )
  def k(hs_hbm, hf_hbm, pos_hbm, embs_hbm, embf_hbm, pidx, rows, sem):
    c = lax.axis_index("c")
    s = lax.axis_index("s")
    base = s * _PPT
    pltpu.sync_copy(pos_hbm.at[pl.ds(base, _PPT)], pidx)

    @pl.when(c == 0)
    def _():
      pltpu.async_copy(hs_hbm.at[pidx], rows, sem).wait()
      pltpu.sync_copy(rows, embs_hbm.at[pl.ds(base, _PPT)])

    @pl.when(c == 1)
    def _():
      pltpu.async_copy(hf_hbm.at[pidx], rows, sem).wait()
      pltpu.sync_copy(rows, embf_hbm.at[pl.ds(base, _PPT)])

  return k(hs, hf, pos)


def _tc_mlp_body(es_ref, ef_ref, w1s, w1f, b1r, w2r, b2r, w3r, b3r, out_ref):
  h = jnp.maximum(
      jnp.dot(es_ref[...], w1s[...], preferred_element_type=jnp.float32)
      + jnp.dot(ef_ref[...], w1f[...], preferred_element_type=jnp.float32)
      + b1r[...], 0.0)
  h = jnp.maximum(
      jnp.dot(h, w2r[...], preferred_element_type=jnp.float32) + b2r[...], 0.0)
  out_ref[...] = (
      jnp.dot(h, w3r[...], preferred_element_type=jnp.float32) + b3r[...])


def _tc_mlp(embs, embf, w1s, w1f, b1, w2, b2, w3p, b3p):
  return pl.pallas_call(
      _tc_mlp_body,
      out_shape=jax.ShapeDtypeStruct((P, 128), jnp.float32),
  )(embs, embf, w1s, w1f, b1, w2, b2, w3p, b3p)


def kernel(x, edge_index, POs, Wn_s, Wself_s, b_s, Wn_f, Wself_f, b_f,
           W1, b1, W2, b2, W3, b3):
  src = edge_index[0]
  dst = edge_index[1]
  ppt = EPT_PAD - EPT
  gpad = ((jnp.arange(NS * ppt, dtype=jnp.int32) * 13) % N).reshape(NS, ppt)
  spad = (N + (jnp.arange(NS * ppt, dtype=jnp.int32) % 256)).reshape(NS, ppt)

  def tile_pad(a, p):
    return jnp.concatenate([a.reshape(NS, EPT), p], axis=1)

  gidx = jnp.stack([tile_pad(src, gpad),
                    tile_pad(dst, gpad)]).reshape(NC, NS, NSUP, SUP, CH)
  sidx = jnp.stack([tile_pad(dst, spad),
                    tile_pad(src, spad)]).reshape(NC, NS, NSUP, SUP, CH)
  zeros = jnp.zeros((RPT_LAST, D), jnp.float32)

  hs = x
  hf = x
  for l in range(LAYERS):
    aggs, aggf = _sc_agg(hs, hf, gidx, sidx, zeros)
    hs, hf = _tc_dense(aggs, hs, aggf, hf,
                       Wn_s[l], Wself_s[l], b_s[l].reshape(1, D),
                       Wn_f[l], Wself_f[l], b_f[l].reshape(1, D))

  embs, embf = _sc_po_gather(hs, hf, POs)
  w3p = jnp.zeros((MLP_DIM, 128), jnp.float32).at[:, :NACT].set(W3)
  b3p = jnp.zeros((1, 128), jnp.float32).at[:, :NACT].set(b3.reshape(1, NACT))
  y = _tc_mlp(embs, embf, W1[:D], W1[D:], b1.reshape(1, MLP_DIM),
              W2, b2.reshape(1, MLP_DIM), w3p, b3p)
  return y[:, :NACT]


# speedup vs baseline: 2.8104x; 1.0543x over previous
import functools

import jax
import jax.numpy as jnp
from jax import lax
from jax.experimental import pallas as pl
from jax.experimental.pallas import tpu as pltpu
from jax.experimental.pallas import tpu_sc as plsc

N = 10000
E = 320000
D = 128
P = 512
MLP_DIM = 256
NACT = 10
LAYERS = 3

NC = 2
NS = 16
CH = 64
NBUF = 4
SUP = 40
NSUP = 8
NCHUNK = SUP * NSUP
EPT = E // NS
EPT_PAD = NCHUNK * CH
NPAD = N + 256
RPT = 624
RPT_LAST = N - (NS - 1) * RPT


def _sc_agg(hs, hf, gidx, sidx, zeros):
  mesh = plsc.VectorSubcoreMesh(core_axis_name="c", subcore_axis_name="s")

  @functools.partial(
      pl.kernel,
      out_type=[jax.ShapeDtypeStruct((N, D), jnp.float32)] * 2,
      mesh=mesh,
      scratch_types=[
          pltpu.VMEM((SUP, CH), jnp.int32),
          pltpu.VMEM((SUP, CH), jnp.int32),
          [pltpu.VMEM((CH, D), jnp.float32)] * NBUF,
          [pltpu.SemaphoreType.DMA] * NBUF,
          [pltpu.SemaphoreType.DMA] * NBUF,
          pltpu.VMEM_SHARED((NPAD, D), jnp.float32),
      ],
  )
  def k(hs_hbm, hf_hbm, g_hbm, s_hbm, z_hbm, aggs_hbm, aggf_hbm,
        gv, sv, bufs, gsems, ssems, agg_sh):
    c = lax.axis_index("c")
    s = lax.axis_index("s")
    base = s * RPT

    @pl.when(s < NS - 1)
    def _():
      pltpu.sync_copy(z_hbm.at[pl.ds(0, RPT)], agg_sh.at[pl.ds(base, RPT)])

    @pl.when(s == NS - 1)
    def _():
      pltpu.sync_copy(z_hbm.at[pl.ds(0, RPT_LAST)],
                      agg_sh.at[pl.ds(base, RPT_LAST)])

    plsc.subcore_barrier()

    def run(h_hbm):
      grp = (0, 1), (2, 3)

      def gather(j, b):
        pltpu.async_copy(h_hbm.at[gv.at[j]], bufs[b], gsems[b])

      def wait_gather(j, b):
        pltpu.make_async_copy(h_hbm.at[gv.at[j]], bufs[b], gsems[b]).wait()

      def scatter(j, b):
        pltpu.async_copy(bufs[b], agg_sh.at[sv.at[j]], ssems[b], add=True)

      def wait_scatter(j, b):
        pltpu.make_async_copy(bufs[b], agg_sh.at[sv.at[j]], ssems[b]).wait()

      def superstep(sup, _):
        pltpu.sync_copy(g_hbm.at[c, s, sup], gv)
        pltpu.sync_copy(s_hbm.at[c, s, sup], sv)
        for b in range(NBUF):
          gather(b, b)

        def round_pair(i, _):
          j0 = i * NBUF
          for g in range(2):
            for k in range(2):
              b = grp[g][k]
              wait_gather(j0 + 2 * g + k, b)
              scatter(j0 + 2 * g + k, b)
            for k in range(2):
              b = grp[g][k]
              wait_scatter(j0 + 2 * g + k, b)
              gather(j0 + NBUF + 2 * g + k, b)
          return 0

        lax.fori_loop(0, SUP // NBUF - 1, round_pair, 0)
        j0 = SUP - NBUF
        for g in range(2):
          for k in range(2):
            b = grp[g][k]
            wait_gather(j0 + 2 * g + k, b)
            scatter(j0 + 2 * g + k, b)
        for g in range(2):
          for k in range(2):
            b = grp[g][k]
            wait_scatter(j0 + 2 * g + k, b)
        return 0

      lax.fori_loop(0, NSUP, superstep, 0)

    @pl.when(c == 0)
    def _():
      run(hs_hbm)

    @pl.when(c == 1)
    def _():
      run(hf_hbm)

    plsc.subcore_barrier()

    def copy_out(out_hbm):
      @pl.when(s < NS - 1)
      def _():
        pltpu.sync_copy(agg_sh.at[pl.ds(base, RPT)],
                        out_hbm.at[pl.ds(base, RPT)])

      @pl.when(s == NS - 1)
      def _():
        pltpu.sync_copy(agg_sh.at[pl.ds(base, RPT_LAST)],
                        out_hbm.at[pl.ds(base, RPT_LAST)])

    @pl.when(c == 0)
    def _():
      copy_out(aggs_hbm)

    @pl.when(c == 1)
    def _():
      copy_out(aggf_hbm)

  return k(hs, hf, gidx, sidx, zeros)


_BLK = 1000


def _tc_dense_body(aggs_ref, hs_ref, aggf_ref, hf_ref,
                   wns, wss, bs1, wnf, wsf, bf1, os_ref, of_ref):
  os_ref[...] = jnp.maximum(
      jnp.dot(aggs_ref[...], wns[...], preferred_element_type=jnp.float32)
      + jnp.dot(hs_ref[...], wss[...], preferred_element_type=jnp.float32)
      + bs1[...], 0.0)
  of_ref[...] = jnp.maximum(
      jnp.dot(aggf_ref[...], wnf[...], preferred_element_type=jnp.float32)
      + jnp.dot(hf_ref[...], wsf[...], preferred_element_type=jnp.float32)
      + bf1[...], 0.0)


def _tc_dense(aggs, hs, aggf, hf, wns, wss, bs1, wnf, wsf, bf1):
  row_spec = pl.BlockSpec((_BLK, D), lambda i: (i, 0))
  w_spec = pl.BlockSpec((D, D), lambda i: (0, 0))
  b_spec = pl.BlockSpec((1, D), lambda i: (0, 0))
  return pl.pallas_call(
      _tc_dense_body,
      grid=(N // _BLK,),
      in_specs=[row_spec, row_spec, row_spec, row_spec,
                w_spec, w_spec, b_spec, w_spec, w_spec, b_spec],
      out_specs=[row_spec, row_spec],
      out_shape=[jax.ShapeDtypeStruct((N, D), jnp.float32)] * 2,
  )(aggs, hs, aggf, hf, wns, wss, bs1, wnf, wsf, bf1)


_PPT = P // NS


def _sc_po_gather(hs, hf, pos):
  mesh = plsc.VectorSubcoreMesh(core_axis_name="c", subcore_axis_name="s")

  @functools.partial(
      pl.kernel,
      out_type=[jax.ShapeDtypeStruct((P, D), jnp.float32)] * 2,
      mesh=mesh,
      scratch_types=[
          pltpu.VMEM((_PPT,), jnp.int32),
          pltpu.VMEM((_PPT, D), jnp.float32),
          pltpu.SemaphoreType.DMA,
      ],
  )
  def k(hs_hbm, hf_hbm, pos_hbm, embs_hbm, embf_hbm, pidx, rows, sem):
    c = lax.axis_index("c")
    s = lax.axis_index("s")
    base = s * _PPT
    pltpu.sync_copy(pos_hbm.at[pl.ds(base, _PPT)], pidx)

    @pl.when(c == 0)
    def _():
      pltpu.async_copy(hs_hbm.at[pidx], rows, sem).wait()
      pltpu.sync_copy(rows, embs_hbm.at[pl.ds(base, _PPT)])

    @pl.when(c == 1)
    def _():
      pltpu.async_copy(hf_hbm.at[pidx], rows, sem).wait()
      pltpu.sync_copy(rows, embf_hbm.at[pl.ds(base, _PPT)])

  return k(hs, hf, pos)


def _tc_mlp_body(es_ref, ef_ref, w1s, w1f, b1r, w2r, b2r, w3r, b3r, out_ref):
  h = jnp.maximum(
      jnp.dot(es_ref[...], w1s[...], preferred_element_type=jnp.float32)
      + jnp.dot(ef_ref[...], w1f[...], preferred_element_type=jnp.float32)
      + b1r[...], 0.0)
  h = jnp.maximum(
      jnp.dot(h, w2r[...], preferred_element_type=jnp.float32) + b2r[...], 0.0)
  out_ref[...] = (
      jnp.dot(h, w3r[...], preferred_element_type=jnp.float32) + b3r[...])


def _tc_mlp(embs, embf, w1s, w1f, b1, w2, b2, w3p, b3p):
  return pl.pallas_call(
      _tc_mlp_body,
      out_shape=jax.ShapeDtypeStruct((P, 128), jnp.float32),
  )(embs, embf, w1s, w1f, b1, w2, b2, w3p, b3p)


def kernel(x, edge_index, POs, Wn_s, Wself_s, b_s, Wn_f, Wself_f, b_f,
           W1, b1, W2, b2, W3, b3):
  src = edge_index[0]
  dst = edge_index[1]
  ppt = EPT_PAD - EPT
  gpad = ((jnp.arange(NS * ppt, dtype=jnp.int32) * 13) % N).reshape(NS, ppt)
  spad = (N + (jnp.arange(NS * ppt, dtype=jnp.int32) % 256)).reshape(NS, ppt)

  def tile_pad(a, p):
    return jnp.concatenate([a.reshape(NS, EPT), p], axis=1)

  gidx = jnp.stack([tile_pad(src, gpad),
                    tile_pad(dst, gpad)]).reshape(NC, NS, NSUP, SUP, CH)
  sidx = jnp.stack([tile_pad(dst, spad),
                    tile_pad(src, spad)]).reshape(NC, NS, NSUP, SUP, CH)
  zeros = jnp.zeros((RPT_LAST, D), jnp.float32)

  hs = x
  hf = x
  for l in range(LAYERS):
    aggs, aggf = _sc_agg(hs, hf, gidx, sidx, zeros)
    hs, hf = _tc_dense(aggs, hs, aggf, hf,
                       Wn_s[l], Wself_s[l], b_s[l].reshape(1, D),
                       Wn_f[l], Wself_f[l], b_f[l].reshape(1, D))

  embs, embf = _sc_po_gather(hs, hf, POs)
  w3p = jnp.zeros((MLP_DIM, 128), jnp.float32).at[:, :NACT].set(W3)
  b3p = jnp.zeros((1, 128), jnp.float32).at[:, :NACT].set(b3.reshape(1, NACT))
  y = _tc_mlp(embs, embf, W1[:D], W1[D:], b1.reshape(1, MLP_DIM),
              W2, b2.reshape(1, MLP_DIM), w3p, b3p)
  return y[:, :NACT]


# speedup vs baseline: 2.8792x; 1.0245x over previous
import functools

import jax
import jax.numpy as jnp
from jax import lax
from jax.experimental import pallas as pl
from jax.experimental.pallas import tpu as pltpu
from jax.experimental.pallas import tpu_sc as plsc

N = 10000
E = 320000
D = 128
P = 512
MLP_DIM = 256
NACT = 10
LAYERS = 3

NC = 2
NS = 16
CH = 40
NBUF = 4
SUP = 100
NSUP = 5
NCHUNK = SUP * NSUP
EPT = E // NS
NPAD = N
RPT = 624
RPT_LAST = N - (NS - 1) * RPT


def _sc_agg(hs, hf, srcv, dstv, zeros):
  mesh = plsc.VectorSubcoreMesh(core_axis_name="c", subcore_axis_name="s")

  @functools.partial(
      pl.kernel,
      out_type=[jax.ShapeDtypeStruct((N, D), jnp.float32)] * 2,
      mesh=mesh,
      scratch_types=[
          pltpu.VMEM((SUP, CH), jnp.int32),
          pltpu.VMEM((SUP, CH), jnp.int32),
          [pltpu.VMEM((CH, D), jnp.float32)] * NBUF,
          [pltpu.SemaphoreType.DMA] * NBUF,
          [pltpu.SemaphoreType.DMA] * NBUF,
          pltpu.VMEM_SHARED((NPAD, D), jnp.float32),
      ],
  )
  def k(hs_hbm, hf_hbm, src_hbm, dst_hbm, z_hbm, aggs_hbm, aggf_hbm,
        gv, sv, bufs, gsems, ssems, agg_sh):
    c = lax.axis_index("c")
    s = lax.axis_index("s")
    base = s * RPT

    @pl.when(s < NS - 1)
    def _():
      pltpu.sync_copy(z_hbm.at[pl.ds(0, RPT)], agg_sh.at[pl.ds(base, RPT)])

    @pl.when(s == NS - 1)
    def _():
      pltpu.sync_copy(z_hbm.at[pl.ds(0, RPT_LAST)],
                      agg_sh.at[pl.ds(base, RPT_LAST)])

    plsc.subcore_barrier()

    def run(h_hbm, g_hbm, s_hbm):
      grp = (0, 1), (2, 3)

      def gather(j, b):
        pltpu.async_copy(h_hbm.at[gv.at[j]], bufs[b], gsems[b])

      def wait_gather(j, b):
        pltpu.make_async_copy(h_hbm.at[gv.at[j]], bufs[b], gsems[b]).wait()

      def scatter(j, b):
        pltpu.async_copy(bufs[b], agg_sh.at[sv.at[j]], ssems[b], add=True)

      def wait_scatter(j, b):
        pltpu.make_async_copy(bufs[b], agg_sh.at[sv.at[j]], ssems[b]).wait()

      def superstep(sup, _):
        pltpu.sync_copy(g_hbm.at[s, sup], gv)
        pltpu.sync_copy(s_hbm.at[s, sup], sv)
        for b in range(NBUF):
          gather(b, b)

        def round_pair(i, _):
          j0 = i * NBUF
          for g in range(2):
            for k in range(2):
              b = grp[g][k]
              wait_gather(j0 + 2 * g + k, b)
              scatter(j0 + 2 * g + k, b)
            for k in range(2):
              b = grp[g][k]
              wait_scatter(j0 + 2 * g + k, b)
              gather(j0 + NBUF + 2 * g + k, b)
          return 0

        lax.fori_loop(0, SUP // NBUF - 1, round_pair, 0)
        j0 = SUP - NBUF
        for g in range(2):
          for k in range(2):
            b = grp[g][k]
            wait_gather(j0 + 2 * g + k, b)
            scatter(j0 + 2 * g + k, b)
        for g in range(2):
          for k in range(2):
            b = grp[g][k]
            wait_scatter(j0 + 2 * g + k, b)
        return 0

      lax.fori_loop(0, NSUP, superstep, 0)

    @pl.when(c == 0)
    def _():
      run(hs_hbm, src_hbm, dst_hbm)

    @pl.when(c == 1)
    def _():
      run(hf_hbm, dst_hbm, src_hbm)

    plsc.subcore_barrier()

    def copy_out(out_hbm):
      @pl.when(s < NS - 1)
      def _():
        pltpu.sync_copy(agg_sh.at[pl.ds(base, RPT)],
                        out_hbm.at[pl.ds(base, RPT)])

      @pl.when(s == NS - 1)
      def _():
        pltpu.sync_copy(agg_sh.at[pl.ds(base, RPT_LAST)],
                        out_hbm.at[pl.ds(base, RPT_LAST)])

    @pl.when(c == 0)
    def _():
      copy_out(aggs_hbm)

    @pl.when(c == 1)
    def _():
      copy_out(aggf_hbm)

  return k(hs, hf, srcv, dstv, zeros)


_BLK = 1000


def _tc_dense_body(aggs_ref, hs_ref, aggf_ref, hf_ref,
                   wns, wss, bs1, wnf, wsf, bf1, os_ref, of_ref):
  os_ref[...] = jnp.maximum(
      jnp.dot(aggs_ref[...], wns[...], preferred_element_type=jnp.float32)
      + jnp.dot(hs_ref[...], wss[...], preferred_element_type=jnp.float32)
      + bs1[...], 0.0)
  of_ref[...] = jnp.maximum(
      jnp.dot(aggf_ref[...], wnf[...], preferred_element_type=jnp.float32)
      + jnp.dot(hf_ref[...], wsf[...], preferred_element_type=jnp.float32)
      + bf1[...], 0.0)


def _tc_dense(aggs, hs, aggf, hf, wns, wss, bs1, wnf, wsf, bf1):
  row_spec = pl.BlockSpec((_BLK, D), lambda i: (i, 0))
  w_spec = pl.BlockSpec((D, D), lambda i: (0, 0))
  b_spec = pl.BlockSpec((1, D), lambda i: (0, 0))
  return pl.pallas_call(
      _tc_dense_body,
      grid=(N // _BLK,),
      in_specs=[row_spec, row_spec, row_spec, row_spec,
                w_spec, w_spec, b_spec, w_spec, w_spec, b_spec],
      out_specs=[row_spec, row_spec],
      out_shape=[jax.ShapeDtypeStruct((N, D), jnp.float32)] * 2,
  )(aggs, hs, aggf, hf, wns, wss, bs1, wnf, wsf, bf1)


_PPT = P // NS


def _sc_po_gather(hs, hf, pos):
  mesh = plsc.VectorSubcoreMesh(core_axis_name="c", subcore_axis_name="s")

  @functools.partial(
      pl.kernel,
      out_type=[jax.ShapeDtypeStruct((P, D), jnp.float32)] * 2,
      mesh=mesh,
      scratch_types=[
          pltpu.VMEM((_PPT,), jnp.int32),
          pltpu.VMEM((_PPT, D), jnp.float32),
          pltpu.SemaphoreType.DMA,
      ],
  )
  def k(hs_hbm, hf_hbm, pos_hbm, embs_hbm, embf_hbm, pidx, rows, sem):
    c = lax.axis_index("c")
    s = lax.axis_index("s")
    base = s * _PPT
    pltpu.sync_copy(pos_hbm.at[pl.ds(base, _PPT)], pidx)

    @pl.when(c == 0)
    def _():
      pltpu.async_copy(hs_hbm.at[pidx], rows, sem).wait()
      pltpu.sync_copy(rows, embs_hbm.at[pl.ds(base, _PPT)])

    @pl.when(c == 1)
    def _():
      pltpu.async_copy(hf_hbm.at[pidx], rows, sem).wait()
      pltpu.sync_copy(rows, embf_hbm.at[pl.ds(base, _PPT)])

  return k(hs, hf, pos)


def _tc_mlp_body(es_ref, ef_ref, w1s, w1f, b1r, w2r, b2r, w3r, b3r, out_ref):
  h = jnp.maximum(
      jnp.dot(es_ref[...], w1s[...], preferred_element_type=jnp.float32)
      + jnp.dot(ef_ref[...], w1f[...], preferred_element_type=jnp.float32)
      + b1r[...], 0.0)
  h = jnp.maximum(
      jnp.dot(h, w2r[...], preferred_element_type=jnp.float32) + b2r[...], 0.0)
  out_ref[...] = (
      jnp.dot(h, w3r[...], preferred_element_type=jnp.float32) + b3r[...])


def _tc_mlp(embs, embf, w1s, w1f, b1, w2, b2, w3p, b3p):
  return pl.pallas_call(
      _tc_mlp_body,
      out_shape=jax.ShapeDtypeStruct((P, 128), jnp.float32),
  )(embs, embf, w1s, w1f, b1, w2, b2, w3p, b3p)


def kernel(x, edge_index, POs, Wn_s, Wself_s, b_s, Wn_f, Wself_f, b_f,
           W1, b1, W2, b2, W3, b3):
  srcv = edge_index[0].reshape(NS, NSUP, SUP, CH)
  dstv = edge_index[1].reshape(NS, NSUP, SUP, CH)
  zeros = jnp.zeros((RPT_LAST, D), jnp.float32)

  hs = x
  hf = x
  for l in range(LAYERS):
    aggs, aggf = _sc_agg(hs, hf, srcv, dstv, zeros)
    hs, hf = _tc_dense(aggs, hs, aggf, hf,
                       Wn_s[l], Wself_s[l], b_s[l].reshape(1, D),
                       Wn_f[l], Wself_f[l], b_f[l].reshape(1, D))

  embs, embf = _sc_po_gather(hs, hf, POs)
  w3p = jnp.zeros((MLP_DIM, 128), jnp.float32).at[:, :NACT].set(W3)
  b3p = jnp.zeros((1, 128), jnp.float32).at[:, :NACT].set(b3.reshape(1, NACT))
  y = _tc_mlp(embs, embf, W1[:D], W1[D:], b1.reshape(1, MLP_DIM),
              W2, b2.reshape(1, MLP_DIM), w3p, b3p)
  return y[:, :NACT]


# speedup vs baseline: 2.9029x; 1.0082x over previous
import functools

import jax
import jax.numpy as jnp
from jax import lax
from jax.experimental import pallas as pl
from jax.experimental.pallas import tpu as pltpu
from jax.experimental.pallas import tpu_sc as plsc

N = 10000
E = 320000
D = 128
P = 512
MLP_DIM = 256
NACT = 10
LAYERS = 3

NC = 2
NS = 16
CH = 40
NBUF = 4
SUP = 100
NSUP = 5
NCHUNK = SUP * NSUP
EPT = E // NS
NPAD = N
RPT = 624
RPT_LAST = N - (NS - 1) * RPT


def _sc_agg(hs, hf, edges, zeros):
  mesh = plsc.VectorSubcoreMesh(core_axis_name="c", subcore_axis_name="s")

  @functools.partial(
      pl.kernel,
      out_type=[jax.ShapeDtypeStruct((N, D), jnp.float32)] * 2,
      mesh=mesh,
      scratch_types=[
          pltpu.VMEM((SUP, CH), jnp.int32),
          pltpu.VMEM((SUP, CH), jnp.int32),
          [pltpu.VMEM((CH, D), jnp.float32)] * NBUF,
          [pltpu.SemaphoreType.DMA] * NBUF,
          [pltpu.SemaphoreType.DMA] * NBUF,
          pltpu.VMEM_SHARED((NPAD, D), jnp.float32),
      ],
  )
  def k(hs_hbm, hf_hbm, e_hbm, z_hbm, aggs_hbm, aggf_hbm,
        gv, sv, bufs, gsems, ssems, agg_sh):
    c = lax.axis_index("c")
    s = lax.axis_index("s")
    base = s * RPT

    @pl.when(s < NS - 1)
    def _():
      pltpu.sync_copy(z_hbm.at[pl.ds(0, RPT)], agg_sh.at[pl.ds(base, RPT)])

    @pl.when(s == NS - 1)
    def _():
      pltpu.sync_copy(z_hbm.at[pl.ds(0, RPT_LAST)],
                      agg_sh.at[pl.ds(base, RPT_LAST)])

    plsc.subcore_barrier()

    def run(h_hbm, ge, se):
      grp = (0, 1), (2, 3)

      def gather(j, b):
        pltpu.async_copy(h_hbm.at[gv.at[j]], bufs[b], gsems[b])

      def wait_gather(j, b):
        pltpu.make_async_copy(h_hbm.at[gv.at[j]], bufs[b], gsems[b]).wait()

      def scatter(j, b):
        pltpu.async_copy(bufs[b], agg_sh.at[sv.at[j]], ssems[b], add=True)

      def wait_scatter(j, b):
        pltpu.make_async_copy(bufs[b], agg_sh.at[sv.at[j]], ssems[b]).wait()

      def superstep(sup, _):
        pltpu.sync_copy(e_hbm.at[ge, s, sup], gv)
        pltpu.sync_copy(e_hbm.at[se, s, sup], sv)
        for b in range(NBUF):
          gather(b, b)

        def round_pair(i, _):
          j0 = i * NBUF
          for g in range(2):
            for k in range(2):
              b = grp[g][k]
              wait_gather(j0 + 2 * g + k, b)
              scatter(j0 + 2 * g + k, b)
            for k in range(2):
              b = grp[g][k]
              wait_scatter(j0 + 2 * g + k, b)
              gather(j0 + NBUF + 2 * g + k, b)
          return 0

        lax.fori_loop(0, SUP // NBUF - 1, round_pair, 0)
        j0 = SUP - NBUF
        for g in range(2):
          for k in range(2):
            b = grp[g][k]
            wait_gather(j0 + 2 * g + k, b)
            scatter(j0 + 2 * g + k, b)
        for g in range(2):
          for k in range(2):
            b = grp[g][k]
            wait_scatter(j0 + 2 * g + k, b)
        return 0

      lax.fori_loop(0, NSUP, superstep, 0)

    @pl.when(c == 0)
    def _():
      run(hs_hbm, 0, 1)

    @pl.when(c == 1)
    def _():
      run(hf_hbm, 1, 0)

    plsc.subcore_barrier()

    def copy_out(out_hbm):
      @pl.when(s < NS - 1)
      def _():
        pltpu.sync_copy(agg_sh.at[pl.ds(base, RPT)],
                        out_hbm.at[pl.ds(base, RPT)])

      @pl.when(s == NS - 1)
      def _():
        pltpu.sync_copy(agg_sh.at[pl.ds(base, RPT_LAST)],
                        out_hbm.at[pl.ds(base, RPT_LAST)])

    @pl.when(c == 0)
    def _():
      copy_out(aggs_hbm)

    @pl.when(c == 1)
    def _():
      copy_out(aggf_hbm)

  return k(hs, hf, edges, zeros)


_BLK = 1000


def _tc_dense_body(aggs_ref, hs_ref, aggf_ref, hf_ref,
                   wns, wss, bs1, wnf, wsf, bf1, os_ref, of_ref):
  os_ref[...] = jnp.maximum(
      jnp.dot(aggs_ref[...], wns[0], preferred_element_type=jnp.float32)
      + jnp.dot(hs_ref[...], wss[0], preferred_element_type=jnp.float32)
      + bs1[0], 0.0)
  of_ref[...] = jnp.maximum(
      jnp.dot(aggf_ref[...], wnf[0], preferred_element_type=jnp.float32)
      + jnp.dot(hf_ref[...], wsf[0], preferred_element_type=jnp.float32)
      + bf1[0], 0.0)


def _tc_dense(aggs, hs, aggf, hf, wns, wss, bs, wnf, wsf, bf, l):
  row_spec = pl.BlockSpec((_BLK, D), lambda i: (i, 0))
  w_spec = pl.BlockSpec((1, D, D), lambda i: (l, 0, 0))
  b_spec = pl.BlockSpec((1, 1, D), lambda i: (l, 0, 0))
  return pl.pallas_call(
      _tc_dense_body,
      grid=(N // _BLK,),
      in_specs=[row_spec, row_spec, row_spec, row_spec,
                w_spec, w_spec, b_spec, w_spec, w_spec, b_spec],
      out_specs=[row_spec, row_spec],
      out_shape=[jax.ShapeDtypeStruct((N, D), jnp.float32)] * 2,
  )(aggs, hs, aggf, hf, wns, wss, bs.reshape(LAYERS, 1, D),
    wnf, wsf, bf.reshape(LAYERS, 1, D))


_PPT = P // NS


def _sc_po_gather(hs, hf, pos):
  mesh = plsc.VectorSubcoreMesh(core_axis_name="c", subcore_axis_name="s")

  @functools.partial(
      pl.kernel,
      out_type=[jax.ShapeDtypeStruct((P, D), jnp.float32)] * 2,
      mesh=mesh,
      scratch_types=[
          pltpu.VMEM((_PPT,), jnp.int32),
          pltpu.VMEM((_PPT, D), jnp.float32),
          pltpu.SemaphoreType.DMA,
      ],
  )
  def k(hs_hbm, hf_hbm, pos_hbm, embs_hbm, embf_hbm, pidx, rows, sem):
    c = lax.axis_index("c")
    s = lax.axis_index("s")
    base = s * _PPT
    pltpu.sync_copy(pos_hbm.at[pl.ds(base, _PPT)], pidx)

    @pl.when(c == 0)
    def _():
      pltpu.async_copy(hs_hbm.at[pidx], rows, sem).wait()
      pltpu.sync_copy(rows, embs_hbm.at[pl.ds(base, _PPT)])

    @pl.when(c == 1)
    def _():
      pltpu.async_copy(hf_hbm.at[pidx], rows, sem).wait()
      pltpu.sync_copy(rows, embf_hbm.at[pl.ds(base, _PPT)])

  return k(hs, hf, pos)


def _tc_mlp_body(es_ref, ef_ref, w1s, w1f, b1r, w2r, b2r, w3r, b3r, out_ref):
  h = jnp.maximum(
      jnp.dot(es_ref[...], w1s[...], preferred_element_type=jnp.float32)
      + jnp.dot(ef_ref[...], w1f[...], preferred_element_type=jnp.float32)
      + b1r[...], 0.0)
  h = jnp.maximum(
      jnp.dot(h, w2r[...], preferred_element_type=jnp.float32) + b2r[...], 0.0)
  out_ref[...] = (
      jnp.dot(h, w3r[...], preferred_element_type=jnp.float32) + b3r[...])


def _tc_mlp(embs, embf, w1, b1, w2, b2, w3, b3):
  full = lambda *shape: pl.BlockSpec(shape, lambda i: (0,) * len(shape))
  return pl.pallas_call(
      _tc_mlp_body,
      grid=(1,),
      in_specs=[full(P, D), full(P, D),
                pl.BlockSpec((D, MLP_DIM), lambda i: (0, 0)),
                pl.BlockSpec((D, MLP_DIM), lambda i: (1, 0)),
                full(1, MLP_DIM), full(MLP_DIM, MLP_DIM), full(1, MLP_DIM),
                full(MLP_DIM, NACT), full(1, NACT)],
      out_specs=full(P, NACT),
      out_shape=jax.ShapeDtypeStruct((P, NACT), jnp.float32),
  )(embs, embf, w1, w1, b1.reshape(1, MLP_DIM), w2, b2.reshape(1, MLP_DIM),
    w3, b3.reshape(1, NACT))


def kernel(x, edge_index, POs, Wn_s, Wself_s, b_s, Wn_f, Wself_f, b_f,
           W1, b1, W2, b2, W3, b3):
  edges = edge_index.reshape(2, NS, NSUP, SUP, CH)
  zeros = jnp.zeros((RPT_LAST, D), jnp.float32)

  hs = x
  hf = x
  for l in range(LAYERS):
    aggs, aggf = _sc_agg(hs, hf, edges, zeros)
    hs, hf = _tc_dense(aggs, hs, aggf, hf,
                       Wn_s, Wself_s, b_s, Wn_f, Wself_f, b_f, l)

  embs, embf = _sc_po_gather(hs, hf, POs)
  return _tc_mlp(embs, embf, W1, b1, W2, b2, W3, b3)


# speedup vs baseline: 2.9511x; 1.0166x over previous
import functools

import jax
import jax.numpy as jnp
from jax import lax
from jax.experimental import pallas as pl
from jax.experimental.pallas import tpu as pltpu
from jax.experimental.pallas import tpu_sc as plsc

N = 10000
E = 320000
D = 128
P = 512
MLP_DIM = 256
NACT = 10
LAYERS = 3

NC = 2
NS = 16
CH = 40
NBUF = 4
SUP = 100
NSUP = 5
NCHUNK = SUP * NSUP
EPT = E // NS
NPAD = N
RPT = 624
RPT_LAST = N - (NS - 1) * RPT


def _sc_agg(hs, hf, edges, zeros):
  mesh = plsc.VectorSubcoreMesh(core_axis_name="c", subcore_axis_name="s")

  @functools.partial(
      pl.kernel,
      out_type=[jax.ShapeDtypeStruct((N, D), jnp.float32)] * 2,
      mesh=mesh,
      scratch_types=[
          pltpu.VMEM((SUP * CH,), jnp.int32),
          pltpu.VMEM((SUP * CH,), jnp.int32),
          [pltpu.VMEM((CH, D), jnp.float32)] * NBUF,
          [pltpu.SemaphoreType.DMA] * NBUF,
          [pltpu.SemaphoreType.DMA] * NBUF,
          pltpu.VMEM_SHARED((NPAD, D), jnp.float32),
      ],
  )
  def k(hs_hbm, hf_hbm, se_hbm, de_hbm, z_hbm, aggs_hbm, aggf_hbm,
        gv, sv, bufs, gsems, ssems, agg_sh):
    c = lax.axis_index("c")
    s = lax.axis_index("s")
    base = s * RPT

    @pl.when(s < NS - 1)
    def _():
      pltpu.sync_copy(z_hbm.at[pl.ds(0, RPT)], agg_sh.at[pl.ds(base, RPT)])

    @pl.when(s == NS - 1)
    def _():
      pltpu.sync_copy(z_hbm.at[pl.ds(0, RPT_LAST)],
                      agg_sh.at[pl.ds(base, RPT_LAST)])

    plsc.subcore_barrier()

    def run(h_hbm, ge, se):
      grp = (0, 1), (2, 3)

      def gather(j, b):
        pltpu.async_copy(h_hbm.at[gv.at[pl.ds(j * CH, CH)]], bufs[b],
                         gsems[b])

      def wait_gather(j, b):
        pltpu.make_async_copy(h_hbm.at[gv.at[pl.ds(j * CH, CH)]], bufs[b],
                              gsems[b]).wait()

      def scatter(j, b):
        pltpu.async_copy(bufs[b], agg_sh.at[sv.at[pl.ds(j * CH, CH)]],
                         ssems[b], add=True)

      def wait_scatter(j, b):
        pltpu.make_async_copy(bufs[b], agg_sh.at[sv.at[pl.ds(j * CH, CH)]],
                              ssems[b]).wait()

      def superstep(sup, _):
        off = s * EPT + sup * (SUP * CH)
        pltpu.sync_copy(ge.at[pl.ds(off, SUP * CH)], gv)
        pltpu.sync_copy(se.at[pl.ds(off, SUP * CH)], sv)
        for b in range(NBUF):
          gather(b, b)

        def round_pair(i, _):
          j0 = i * NBUF
          for g in range(2):
            for k in range(2):
              b = grp[g][k]
              wait_gather(j0 + 2 * g + k, b)
              scatter(j0 + 2 * g + k, b)
            for k in range(2):
              b = grp[g][k]
              wait_scatter(j0 + 2 * g + k, b)
              gather(j0 + NBUF + 2 * g + k, b)
          return 0

        lax.fori_loop(0, SUP // NBUF - 1, round_pair, 0)
        j0 = SUP - NBUF
        for g in range(2):
          for k in range(2):
            b = grp[g][k]
            wait_gather(j0 + 2 * g + k, b)
            scatter(j0 + 2 * g + k, b)
        for g in range(2):
          for k in range(2):
            b = grp[g][k]
            wait_scatter(j0 + 2 * g + k, b)
        return 0

      lax.fori_loop(0, NSUP, superstep, 0)

    @pl.when(c == 0)
    def _():
      run(hs_hbm, se_hbm, de_hbm)

    @pl.when(c == 1)
    def _():
      run(hf_hbm, de_hbm, se_hbm)

    plsc.subcore_barrier()

    def copy_out(out_hbm):
      @pl.when(s < NS - 1)
      def _():
        pltpu.sync_copy(agg_sh.at[pl.ds(base, RPT)],
                        out_hbm.at[pl.ds(base, RPT)])

      @pl.when(s == NS - 1)
      def _():
        pltpu.sync_copy(agg_sh.at[pl.ds(base, RPT_LAST)],
                        out_hbm.at[pl.ds(base, RPT_LAST)])

    @pl.when(c == 0)
    def _():
      copy_out(aggs_hbm)

    @pl.when(c == 1)
    def _():
      copy_out(aggf_hbm)

  return k(hs, hf, edges[0], edges[1], zeros)


_BLK = 1000


_PREC = jax.lax.Precision.DEFAULT


def _tc_dense_body(aggs_ref, hs_ref, aggf_ref, hf_ref,
                   wns, wss, bs1, wnf, wsf, bf1, os_ref, of_ref):
  os_ref[...] = jnp.maximum(
      jnp.dot(aggs_ref[...], wns[0], preferred_element_type=jnp.float32,
              precision=_PREC)
      + jnp.dot(hs_ref[...], wss[0], preferred_element_type=jnp.float32,
                precision=_PREC)
      + bs1[0], 0.0)
  of_ref[...] = jnp.maximum(
      jnp.dot(aggf_ref[...], wnf[0], preferred_element_type=jnp.float32,
              precision=_PREC)
      + jnp.dot(hf_ref[...], wsf[0], preferred_element_type=jnp.float32,
                precision=_PREC)
      + bf1[0], 0.0)


def _tc_dense(aggs, hs, aggf, hf, wns, wss, bs, wnf, wsf, bf, l):
  row_spec = pl.BlockSpec((_BLK, D), lambda i: (i, 0))
  w_spec = pl.BlockSpec((1, D, D), lambda i: (l, 0, 0))
  b_spec = pl.BlockSpec((1, 1, D), lambda i: (l, 0, 0))
  return pl.pallas_call(
      _tc_dense_body,
      grid=(N // _BLK,),
      in_specs=[row_spec, row_spec, row_spec, row_spec,
                w_spec, w_spec, b_spec, w_spec, w_spec, b_spec],
      out_specs=[row_spec, row_spec],
      out_shape=[jax.ShapeDtypeStruct((N, D), jnp.float32)] * 2,
  )(aggs, hs, aggf, hf, wns, wss, bs.reshape(LAYERS, 1, D),
    wnf, wsf, bf.reshape(LAYERS, 1, D))


_PPT = P // NS


def _sc_po_gather(hs, hf, pos):
  mesh = plsc.VectorSubcoreMesh(core_axis_name="c", subcore_axis_name="s")

  @functools.partial(
      pl.kernel,
      out_type=[jax.ShapeDtypeStruct((P, D), jnp.float32)] * 2,
      mesh=mesh,
      scratch_types=[
          pltpu.VMEM((_PPT,), jnp.int32),
          pltpu.VMEM((_PPT, D), jnp.float32),
          pltpu.SemaphoreType.DMA,
      ],
  )
  def k(hs_hbm, hf_hbm, pos_hbm, embs_hbm, embf_hbm, pidx, rows, sem):
    c = lax.axis_index("c")
    s = lax.axis_index("s")
    base = s * _PPT
    pltpu.sync_copy(pos_hbm.at[pl.ds(base, _PPT)], pidx)

    @pl.when(c == 0)
    def _():
      pltpu.async_copy(hs_hbm.at[pidx], rows, sem).wait()
      pltpu.sync_copy(rows, embs_hbm.at[pl.ds(base, _PPT)])

    @pl.when(c == 1)
    def _():
      pltpu.async_copy(hf_hbm.at[pidx], rows, sem).wait()
      pltpu.sync_copy(rows, embf_hbm.at[pl.ds(base, _PPT)])

  return k(hs, hf, pos)


def _tc_mlp_body(es_ref, ef_ref, w1s, w1f, b1r, w2r, b2r, w3r, b3r, out_ref):
  h = jnp.maximum(
      jnp.dot(es_ref[...], w1s[...], preferred_element_type=jnp.float32)
      + jnp.dot(ef_ref[...], w1f[...], preferred_element_type=jnp.float32)
      + b1r[...], 0.0)
  h = jnp.maximum(
      jnp.dot(h, w2r[...], preferred_element_type=jnp.float32) + b2r[...], 0.0)
  out_ref[...] = (
      jnp.dot(h, w3r[...], preferred_element_type=jnp.float32) + b3r[...])


def _tc_mlp(embs, embf, w1, b1, w2, b2, w3, b3):
  full = lambda *shape: pl.BlockSpec(shape, lambda i: (0,) * len(shape))
  return pl.pallas_call(
      _tc_mlp_body,
      grid=(1,),
      in_specs=[full(P, D), full(P, D),
                pl.BlockSpec((D, MLP_DIM), lambda i: (0, 0)),
                pl.BlockSpec((D, MLP_DIM), lambda i: (1, 0)),
                full(1, MLP_DIM), full(MLP_DIM, MLP_DIM), full(1, MLP_DIM),
                full(MLP_DIM, NACT), full(1, NACT)],
      out_specs=full(P, NACT),
      out_shape=jax.ShapeDtypeStruct((P, NACT), jnp.float32),
  )(embs, embf, w1, w1, b1.reshape(1, MLP_DIM), w2, b2.reshape(1, MLP_DIM),
    w3, b3.reshape(1, NACT))


def kernel(x, edge_index, POs, Wn_s, Wself_s, b_s, Wn_f, Wself_f, b_f,
           W1, b1, W2, b2, W3, b3):
  zeros = jnp.zeros((RPT_LAST, D), jnp.float32)

  hs = x
  hf = x
  for l in range(LAYERS):
    aggs, aggf = _sc_agg(hs, hf, edge_index, zeros)
    hs, hf = _tc_dense(aggs, hs, aggf, hf,
                       Wn_s, Wself_s, b_s, Wn_f, Wself_f, b_f, l)

  embs, embf = _sc_po_gather(hs, hf, POs)
  return _tc_mlp(embs, embf, W1, b1, W2, b2, W3, b3)


# speedup vs baseline: 3.0014x; 1.0170x over previous
import functools

import jax
import jax.numpy as jnp
from jax import lax
from jax.experimental import pallas as pl
from jax.experimental.pallas import tpu as pltpu
from jax.experimental.pallas import tpu_sc as plsc

N = 10000
E = 320000
D = 128
P = 512
MLP_DIM = 256
NACT = 10
LAYERS = 3

NC = 2
NS = 16
CH = 40
NBUF = 4
SUP = 100
NSUP = 5
NCHUNK = SUP * NSUP
EPT = E // NS
NPAD = N
RPT = 624
RPT_LAST = N - (NS - 1) * RPT


def _sc_agg(hs, hf, edges, zeros):
  mesh = plsc.VectorSubcoreMesh(core_axis_name="c", subcore_axis_name="s")

  @functools.partial(
      pl.kernel,
      out_type=[jax.ShapeDtypeStruct((N, D), jnp.float32)] * 2,
      mesh=mesh,
      scratch_types=[
          pltpu.VMEM((SUP * CH,), jnp.int32),
          pltpu.VMEM((SUP * CH,), jnp.int32),
          [pltpu.VMEM((CH, D), jnp.float32)] * NBUF,
          [pltpu.SemaphoreType.DMA] * NBUF,
          [pltpu.SemaphoreType.DMA] * NBUF,
          pltpu.VMEM_SHARED((NPAD, D), jnp.float32),
      ],
  )
  def k(hs_hbm, hf_hbm, se_hbm, de_hbm, z_hbm, aggs_hbm, aggf_hbm,
        gv, sv, bufs, gsems, ssems, agg_sh):
    c = lax.axis_index("c")
    s = lax.axis_index("s")
    base = s * RPT

    @pl.when(s < NS - 1)
    def _():
      pltpu.sync_copy(z_hbm.at[pl.ds(0, RPT)], agg_sh.at[pl.ds(base, RPT)])

    @pl.when(s == NS - 1)
    def _():
      pltpu.sync_copy(z_hbm.at[pl.ds(0, RPT_LAST)],
                      agg_sh.at[pl.ds(base, RPT_LAST)])

    plsc.subcore_barrier()

    def run(h_hbm, ge, se):
      grp = (0, 1), (2, 3)

      def gather(j, b):
        pltpu.async_copy(h_hbm.at[gv.at[pl.ds(j * CH, CH)]], bufs[b],
                         gsems[b])

      def wait_gather(j, b):
        pltpu.make_async_copy(h_hbm.at[gv.at[pl.ds(j * CH, CH)]], bufs[b],
                              gsems[b]).wait()

      def scatter(j, b):
        pltpu.async_copy(bufs[b], agg_sh.at[sv.at[pl.ds(j * CH, CH)]],
                         ssems[b], add=True)

      def wait_scatter(j, b):
        pltpu.make_async_copy(bufs[b], agg_sh.at[sv.at[pl.ds(j * CH, CH)]],
                              ssems[b]).wait()

      def superstep(sup, _):
        off = s * EPT + sup * (SUP * CH)
        pltpu.sync_copy(ge.at[pl.ds(off, SUP * CH)], gv)
        pltpu.sync_copy(se.at[pl.ds(off, SUP * CH)], sv)
        for b in range(NBUF):
          gather(b, b)

        def round_pair(i, _):
          j0 = i * NBUF
          for g in range(2):
            for k in range(2):
              b = grp[g][k]
              wait_gather(j0 + 2 * g + k, b)
              scatter(j0 + 2 * g + k, b)
            for k in range(2):
              b = grp[g][k]
              wait_scatter(j0 + 2 * g + k, b)
              gather(j0 + NBUF + 2 * g + k, b)
          return 0

        lax.fori_loop(0, SUP // NBUF - 1, round_pair, 0)
        j0 = SUP - NBUF
        for g in range(2):
          for k in range(2):
            b = grp[g][k]
            wait_gather(j0 + 2 * g + k, b)
            scatter(j0 + 2 * g + k, b)
        for g in range(2):
          for k in range(2):
            b = grp[g][k]
            wait_scatter(j0 + 2 * g + k, b)
        return 0

      lax.fori_loop(0, NSUP, superstep, 0)

    @pl.when(c == 0)
    def _():
      run(hs_hbm, se_hbm, de_hbm)

    @pl.when(c == 1)
    def _():
      run(hf_hbm, de_hbm, se_hbm)

    plsc.subcore_barrier()

    def copy_out(out_hbm):
      @pl.when(s < NS - 1)
      def _():
        pltpu.sync_copy(agg_sh.at[pl.ds(base, RPT)],
                        out_hbm.at[pl.ds(base, RPT)])

      @pl.when(s == NS - 1)
      def _():
        pltpu.sync_copy(agg_sh.at[pl.ds(base, RPT_LAST)],
                        out_hbm.at[pl.ds(base, RPT_LAST)])

    @pl.when(c == 0)
    def _():
      copy_out(aggs_hbm)

    @pl.when(c == 1)
    def _():
      copy_out(aggf_hbm)

  return k(hs, hf, edges[0], edges[1], zeros)


_BLK = 1000


_PREC = jax.lax.Precision.DEFAULT


def _tc_dense_body(aggs_ref, hs_ref, aggf_ref, hf_ref,
                   wns, wss, bs1, wnf, wsf, bf1, os_ref, of_ref):
  os_ref[...] = jnp.maximum(
      jnp.dot(aggs_ref[...], wns[0], preferred_element_type=jnp.float32,
              precision=_PREC)
      + jnp.dot(hs_ref[...], wss[0], preferred_element_type=jnp.float32,
                precision=_PREC)
      + bs1[0], 0.0)
  of_ref[...] = jnp.maximum(
      jnp.dot(aggf_ref[...], wnf[0], preferred_element_type=jnp.float32,
              precision=_PREC)
      + jnp.dot(hf_ref[...], wsf[0], preferred_element_type=jnp.float32,
                precision=_PREC)
      + bf1[0], 0.0)


def _tc_dense(aggs, hs, aggf, hf, wns, wss, bs, wnf, wsf, bf, l):
  row_spec = pl.BlockSpec((_BLK, D), lambda i: (i, 0))
  w_spec = pl.BlockSpec((1, D, D), lambda i: (l, 0, 0))
  b_spec = pl.BlockSpec((1, 1, D), lambda i: (l, 0, 0))
  return pl.pallas_call(
      _tc_dense_body,
      grid=(N // _BLK,),
      in_specs=[row_spec, row_spec, row_spec, row_spec,
                w_spec, w_spec, b_spec, w_spec, w_spec, b_spec],
      out_specs=[row_spec, row_spec],
      out_shape=[jax.ShapeDtypeStruct((N, D), jnp.float32)] * 2,
  )(aggs, hs, aggf, hf, wns, wss, bs.reshape(LAYERS, 1, D),
    wnf, wsf, bf.reshape(LAYERS, 1, D))


_PPT = P // NS


def _sc_po_gather4(aggs, hs, aggf, hf, pos):
  mesh = plsc.VectorSubcoreMesh(core_axis_name="c", subcore_axis_name="s")

  @functools.partial(
      pl.kernel,
      out_type=[jax.ShapeDtypeStruct((P, D), jnp.float32)] * 4,
      mesh=mesh,
      scratch_types=[
          pltpu.VMEM((_PPT,), jnp.int32),
          pltpu.VMEM((_PPT, D), jnp.float32),
          pltpu.VMEM((_PPT, D), jnp.float32),
          pltpu.SemaphoreType.DMA,
          pltpu.SemaphoreType.DMA,
      ],
  )
  def k(aggs_hbm, hs_hbm, aggf_hbm, hf_hbm, pos_hbm,
        oas_hbm, ohs_hbm, oaf_hbm, ohf_hbm, pidx, rows0, rows1, sem0, sem1):
    c = lax.axis_index("c")
    s = lax.axis_index("s")
    base = s * _PPT
    pltpu.sync_copy(pos_hbm.at[pl.ds(base, _PPT)], pidx)

    def gather2(a_hbm, h_hbm, oa_hbm, oh_hbm):
      pltpu.async_copy(a_hbm.at[pidx], rows0, sem0)
      pltpu.async_copy(h_hbm.at[pidx], rows1, sem1)
      pltpu.make_async_copy(a_hbm.at[pidx], rows0, sem0).wait()
      pltpu.sync_copy(rows0, oa_hbm.at[pl.ds(base, _PPT)])
      pltpu.make_async_copy(h_hbm.at[pidx], rows1, sem1).wait()
      pltpu.sync_copy(rows1, oh_hbm.at[pl.ds(base, _PPT)])

    @pl.when(c == 0)
    def _():
      gather2(aggs_hbm, hs_hbm, oas_hbm, ohs_hbm)

    @pl.when(c == 1)
    def _():
      gather2(aggf_hbm, hf_hbm, oaf_hbm, ohf_hbm)

  return k(aggs, hs, aggf, hf, pos)


def _tc_head_body(as_ref, hs_ref, af_ref, hf_ref, wns, wss, bs1, wnf, wsf, bf1,
                  w1s, w1f, b1r, w2r, b2r, w3r, b3r, out_ref):
  prec = dict(preferred_element_type=jnp.float32, precision=_PREC)
  es = jnp.maximum(jnp.dot(as_ref[...], wns[0], **prec)
                   + jnp.dot(hs_ref[...], wss[0], **prec) + bs1[0], 0.0)
  ef = jnp.maximum(jnp.dot(af_ref[...], wnf[0], **prec)
                   + jnp.dot(hf_ref[...], wsf[0], **prec) + bf1[0], 0.0)
  h = jnp.maximum(jnp.dot(es, w1s[...], **prec)
                  + jnp.dot(ef, w1f[...], **prec) + b1r[...], 0.0)
  h = jnp.maximum(jnp.dot(h, w2r[...], **prec) + b2r[...], 0.0)
  out_ref[...] = jnp.dot(h, w3r[...], **prec) + b3r[...]


def _tc_head(aggs_p, hs_p, aggf_p, hf_p, wns, wss, bs, wnf, wsf, bf,
             w1, b1, w2, b2, w3, b3):
  l = LAYERS - 1
  full = lambda *shape: pl.BlockSpec(shape, lambda i: (0,) * len(shape))
  wl_spec = pl.BlockSpec((1, D, D), lambda i: (l, 0, 0))
  bl_spec = pl.BlockSpec((1, 1, D), lambda i: (l, 0, 0))
  return pl.pallas_call(
      _tc_head_body,
      grid=(1,),
      in_specs=[full(P, D), full(P, D), full(P, D), full(P, D),
                wl_spec, wl_spec, bl_spec, wl_spec, wl_spec, bl_spec,
                pl.BlockSpec((D, MLP_DIM), lambda i: (0, 0)),
                pl.BlockSpec((D, MLP_DIM), lambda i: (1, 0)),
                full(1, MLP_DIM), full(MLP_DIM, MLP_DIM), full(1, MLP_DIM),
                full(MLP_DIM, NACT), full(1, NACT)],
      out_specs=full(P, NACT),
      out_shape=jax.ShapeDtypeStruct((P, NACT), jnp.float32),
  )(aggs_p, hs_p, aggf_p, hf_p,
    wns, wss, bs.reshape(LAYERS, 1, D), wnf, wsf, bf.reshape(LAYERS, 1, D),
    w1, w1, b1.reshape(1, MLP_DIM), w2, b2.reshape(1, MLP_DIM),
    w3, b3.reshape(1, NACT))


def kernel(x, edge_index, POs, Wn_s, Wself_s, b_s, Wn_f, Wself_f, b_f,
           W1, b1, W2, b2, W3, b3):
  zeros = jnp.zeros((RPT_LAST, D), jnp.float32)

  hs = x
  hf = x
  for l in range(LAYERS - 1):
    aggs, aggf = _sc_agg(hs, hf, edge_index, zeros)
    hs, hf = _tc_dense(aggs, hs, aggf, hf,
                       Wn_s, Wself_s, b_s, Wn_f, Wself_f, b_f, l)

  aggs, aggf = _sc_agg(hs, hf, edge_index, zeros)
  aggs_p, hs_p, aggf_p, hf_p = _sc_po_gather4(aggs, hs, aggf, hf, POs)
  return _tc_head(aggs_p, hs_p, aggf_p, hf_p,
                  Wn_s, Wself_s, b_s, Wn_f, Wself_f, b_f,
                  W1, b1, W2, b2, W3, b3)
